# Initial kernel scaffold; baseline (speedup 1.0000x reference)
import jax
import jax.numpy as jnp
from jax.experimental import pallas as pl


def kernel(aa_seqs, s, s_init, W_in, b_in, W_init, b_init, Wb, bb, W_out, b_out):
    raise NotImplementedError("write your pallas kernel here")


# speedup vs baseline: 1.2508x; 1.2508x over previous
import numpy as np
import jax
import jax.numpy as jnp
from jax.experimental import pallas as pl
from jax.experimental.pallas import tpu as pltpu

_NA = 7
_OUT = _NA * 2
_TILE = 256


def _mlp_body(ids_ref, x_ref, xi_ref, Win_ref, bin_ref, Winit_ref,
              binit_ref, Wb_ref, bb_ref, Wout_ref, bout_ref, out_ref):
    e = pl.program_id(1)
    num_e = pl.num_programs(1)

    x = x_ref[...]
    xi = xi_ref[...]
    a = jnp.dot(jnp.maximum(xi, 0.0), Winit_ref[0],
                preferred_element_type=jnp.float32) + binit_ref[0]
    h = jnp.dot(jnp.maximum(x, 0.0), Win_ref[0],
                preferred_element_type=jnp.float32) + bin_ref[0] + a
    for b in range(2):
        t = jnp.dot(jnp.maximum(h, 0.0), Wb_ref[0, 2 * b],
                    preferred_element_type=jnp.float32) + bb_ref[0, 2 * b]
        t = jnp.dot(jnp.maximum(t, 0.0), Wb_ref[0, 2 * b + 1],
                    preferred_element_type=jnp.float32) + bb_ref[0, 2 * b + 1]
        h = h + t
    o = jnp.dot(jnp.maximum(h, 0.0), Wout_ref[0],
                preferred_element_type=jnp.float32) + bout_ref[0]

    mask = ids_ref[0] == e

    @pl.when(e == 0)
    def _init():
        col = jax.lax.broadcasted_iota(jnp.int32, (_TILE, _OUT), 1)
        out_ref[...] = jnp.where(col % 2 == 0, 1.0, 0.0)

    out_ref[...] = jnp.where(mask, o, out_ref[...])

    @pl.when(e == num_e - 1)
    def _normalize():
        ri = jax.lax.broadcasted_iota(jnp.int32, (_OUT, _OUT), 0)
        ci = jax.lax.broadcasted_iota(jnp.int32, (_OUT, _OUT), 1)
        pair = (ri // 2 == ci // 2).astype(jnp.float32)
        v = out_ref[...]
        n = jnp.sqrt(jnp.dot(v * v, pair, preferred_element_type=jnp.float32))
        out_ref[...] = v / jnp.maximum(n, 1e-12)


def kernel(aa_seqs, s, s_init, W_in, b_in, W_init, b_init, Wb, bb, W_out,
           b_out):
    bs, seq_len, C = s.shape
    T = bs * seq_len
    E = W_in.shape[0]
    CH = W_init.shape[2]

    sf = s.reshape(T, C)
    sif = s_init.reshape(T, C)
    ids3 = aa_seqs.reshape(T // _TILE, _TILE, 1)
    Wb2 = Wb.reshape(E, 4, CH, CH)
    bb2 = bb.reshape(E, 4, CH)
    bin2 = b_in.reshape(E, 1, CH)
    binit2 = b_init.reshape(E, 1, CH)
    bout2 = b_out.reshape(E, 1, _OUT)

    out = pl.pallas_call(
        _mlp_body,
        grid=(T // _TILE, E),
        in_specs=[
            pl.BlockSpec((1, _TILE, 1), lambda t, e: (t, 0, 0)),
            pl.BlockSpec((_TILE, C), lambda t, e: (t, 0)),
            pl.BlockSpec((_TILE, C), lambda t, e: (t, 0)),
            pl.BlockSpec((1, C, CH), lambda t, e: (e, 0, 0)),
            pl.BlockSpec((1, 1, CH), lambda t, e: (e, 0, 0)),
            pl.BlockSpec((1, C, CH), lambda t, e: (e, 0, 0)),
            pl.BlockSpec((1, 1, CH), lambda t, e: (e, 0, 0)),
            pl.BlockSpec((1, 4, CH, CH), lambda t, e: (e, 0, 0, 0)),
            pl.BlockSpec((1, 4, CH), lambda t, e: (e, 0, 0)),
            pl.BlockSpec((1, CH, _OUT), lambda t, e: (e, 0, 0)),
            pl.BlockSpec((1, 1, _OUT), lambda t, e: (e, 0, 0)),
        ],
        out_specs=pl.BlockSpec((_TILE, _OUT), lambda t, e: (t, 0)),
        out_shape=jax.ShapeDtypeStruct((T, _OUT), jnp.float32),
        compiler_params=pltpu.CompilerParams(
            dimension_semantics=("parallel", "arbitrary")),
    )(ids3, sf, sif, W_in, bin2, W_init, binit2, Wb2, bb2, W_out, bout2)

    return out.reshape(bs, seq_len, _NA, 2)


# speedup vs baseline: 1.4838x; 1.1863x over previous
import functools

import jax
import jax.numpy as jnp
from jax import lax
from jax.experimental import pallas as pl
from jax.experimental.pallas import tpu as pltpu
from jax.experimental.pallas import tpu_sc as plsc

_NA = 7
_OUT = _NA * 2
_OUTP = 128
_TILE = 256
_NW = 32
_CHUNK = 96


def _routing(ids, E, T):
    G = T // _TILE + E
    B = G * _TILE
    eids = jnp.arange(E, dtype=jnp.int32)
    oh = (ids[:, None] == eids[None, :]).astype(jnp.int32)
    counts = oh.sum(axis=0)
    rank = jnp.take_along_axis(jnp.cumsum(oh, axis=0) - oh,
                               ids[:, None], axis=1)[:, 0]
    ntiles = (counts + _TILE - 1) // _TILE
    csum = jnp.cumsum(ntiles)
    step_start = csum - ntiles
    expert_of_step = jnp.clip(
        jnp.searchsorted(csum, jnp.arange(G, dtype=jnp.int32), side="right"),
        0, E - 1).astype(jnp.int32)
    pos = (step_start[ids] * _TILE + rank).astype(jnp.int32)
    gidx = jnp.zeros((B,), jnp.int32).at[pos].set(
        jnp.arange(T, dtype=jnp.int32))
    return expert_of_step, pos, gidx, G, B


def _sc_gather2(x, xi, gidx, B, D):
    bpw = B // _NW
    nchunk = bpw // _CHUNK
    idx3 = gidx.reshape(_NW, nchunk, _CHUNK)
    mesh = plsc.VectorSubcoreMesh(core_axis_name="c", subcore_axis_name="s")

    @functools.partial(
        pl.kernel, mesh=mesh,
        out_type=(jax.ShapeDtypeStruct((B, D), jnp.float32),
                  jax.ShapeDtypeStruct((B, D), jnp.float32)),
        scratch_types=[pltpu.VMEM((nchunk, _CHUNK), jnp.int32),
                       pltpu.VMEM((bpw, D), jnp.float32),
                       pltpu.SemaphoreType.DMA])
    def gk(x_hbm, xi_hbm, idx_hbm, ox_hbm, oxi_hbm, idx_v, rows_v, sem):
        wid = lax.axis_index("s") * 2 + lax.axis_index("c")
        base = wid * bpw
        pltpu.sync_copy(idx_hbm.at[wid], idx_v)
        for src, dst in ((x_hbm, ox_hbm), (xi_hbm, oxi_hbm)):
            copies = [
                pltpu.async_copy(src.at[idx_v.at[j]],
                                 rows_v.at[pl.ds(j * _CHUNK, _CHUNK)], sem)
                for j in range(nchunk)
            ]
            for c in copies:
                c.wait()
            pltpu.sync_copy(rows_v, dst.at[pl.ds(base, bpw)])

    return gk(x, xi, idx3)


def _sc_unpermute(osort, pos, T):
    bpw = T // _NW
    idx2 = pos.reshape(_NW, bpw)
    mesh = plsc.VectorSubcoreMesh(core_axis_name="c", subcore_axis_name="s")

    @functools.partial(
        pl.kernel, mesh=mesh,
        out_type=jax.ShapeDtypeStruct((T, _OUTP), jnp.float32),
        scratch_types=[pltpu.VMEM((bpw,), jnp.int32),
                       pltpu.VMEM((bpw, _OUTP), jnp.float32),
                       pltpu.SemaphoreType.DMA])
    def uk(src_hbm, idx_hbm, out_hbm, idx_v, rows_v, sem):
        wid = lax.axis_index("s") * 2 + lax.axis_index("c")
        pltpu.sync_copy(idx_hbm.at[wid], idx_v)
        pltpu.async_copy(src_hbm.at[idx_v], rows_v, sem).wait()
        pltpu.sync_copy(rows_v, out_hbm.at[pl.ds(wid * bpw, bpw)])

    return uk(osort, idx2)


def _mlp_body(e_sref, x_ref, xi_ref, Win_ref, bin_ref, Winit_ref, binit_ref,
              Wb_ref, bb_ref, Wout_ref, bout_ref, out_ref):
    x = x_ref[...]
    xi = xi_ref[...]
    a = jnp.dot(jnp.maximum(xi, 0.0), Winit_ref[0],
                preferred_element_type=jnp.float32) + binit_ref[0]
    h = jnp.dot(jnp.maximum(x, 0.0), Win_ref[0],
                preferred_element_type=jnp.float32) + bin_ref[0] + a
    for b in range(2):
        t = jnp.dot(jnp.maximum(h, 0.0), Wb_ref[0, 2 * b],
                    preferred_element_type=jnp.float32) + bb_ref[0, 2 * b]
        t = jnp.dot(jnp.maximum(t, 0.0), Wb_ref[0, 2 * b + 1],
                    preferred_element_type=jnp.float32) + bb_ref[0, 2 * b + 1]
        h = h + t
    o = jnp.dot(jnp.maximum(h, 0.0), Wout_ref[0],
                preferred_element_type=jnp.float32) + bout_ref[0]
    ri = lax.broadcasted_iota(jnp.int32, (_OUTP, _OUTP), 0)
    ci = lax.broadcasted_iota(jnp.int32, (_OUTP, _OUTP), 1)
    pair = (ri // 2 == ci // 2).astype(jnp.float32)
    n = jnp.sqrt(jnp.dot(o * o, pair, preferred_element_type=jnp.float32))
    out_ref[...] = o / jnp.maximum(n, 1e-12)


def _tc_mlp(eos, xg, xig, W_in, bin2, W_init, binit2, Wb4, bb4, Woutp, boutp,
            G, B, C, CH):
    grid_spec = pltpu.PrefetchScalarGridSpec(
        num_scalar_prefetch=1,
        grid=(G,),
        in_specs=[
            pl.BlockSpec((_TILE, C), lambda g, eref: (g, 0)),
            pl.BlockSpec((_TILE, C), lambda g, eref: (g, 0)),
            pl.BlockSpec((1, C, CH), lambda g, eref: (eref[g], 0, 0)),
            pl.BlockSpec((1, 1, CH), lambda g, eref: (eref[g], 0, 0)),
            pl.BlockSpec((1, C, CH), lambda g, eref: (eref[g], 0, 0)),
            pl.BlockSpec((1, 1, CH), lambda g, eref: (eref[g], 0, 0)),
            pl.BlockSpec((1, 4, CH, CH), lambda g, eref: (eref[g], 0, 0, 0)),
            pl.BlockSpec((1, 4, CH), lambda g, eref: (eref[g], 0, 0)),
            pl.BlockSpec((1, CH, _OUTP), lambda g, eref: (eref[g], 0, 0)),
            pl.BlockSpec((1, 1, _OUTP), lambda g, eref: (eref[g], 0, 0)),
        ],
        out_specs=pl.BlockSpec((_TILE, _OUTP), lambda g, eref: (g, 0)),
    )
    return pl.pallas_call(
        _mlp_body,
        grid_spec=grid_spec,
        out_shape=jax.ShapeDtypeStruct((B, _OUTP), jnp.float32),
        compiler_params=pltpu.CompilerParams(
            dimension_semantics=("arbitrary",)),
    )(eos, xg, xig, W_in, bin2, W_init, binit2, Wb4, bb4, Woutp, boutp)


def kernel(aa_seqs, s, s_init, W_in, b_in, W_init, b_init, Wb, bb, W_out,
           b_out):
    bs, seq_len, C = s.shape
    T = bs * seq_len
    E, _, CH = W_in.shape

    ids = aa_seqs.reshape(T).astype(jnp.int32)
    sf = s.reshape(T, C)
    sif = s_init.reshape(T, C)

    eos, pos, gidx, G, B = _routing(ids, E, T)
    xg, xig = _sc_gather2(sf, sif, gidx, B, C)

    bin2 = b_in.reshape(E, 1, CH)
    binit2 = b_init.reshape(E, 1, CH)
    Wb4 = Wb.reshape(E, 4, CH, CH)
    bb4 = bb.reshape(E, 4, CH)
    Woutp = jnp.zeros((E, CH, _OUTP), W_out.dtype).at[:, :, :_OUT].set(W_out)
    boutp = jnp.zeros((E, 1, _OUTP), b_out.dtype).at[:, 0, :_OUT].set(b_out)

    osort = _tc_mlp(eos, xg, xig, W_in, bin2, W_init, binit2, Wb4, bb4,
                    Woutp, boutp, G, B, C, CH)
    outp = _sc_unpermute(osort, pos, T)
    return outp[:, :_OUT].reshape(bs, seq_len, _NA, 2)
